# Initial kernel scaffold; baseline (speedup 1.0000x reference)
import jax
import jax.numpy as jnp
from jax import lax
from jax.experimental import pallas as pl
from jax.experimental.pallas import tpu as pltpu
from jax.experimental.pallas import tpu_sc as plsc

N = 10000
E = 320000
HID = 128
LAT = 64
VOCAB = 800

NC, NS = 2, 16
NW = NC * NS
EPW = E // NW
C = 80
NCHUNK = EPW // C
RPT = N // NS

f32 = jnp.float32
_MESH = plsc.VectorSubcoreMesh(core_axis_name="c", subcore_axis_name="s",
                               num_cores=NC, num_subcores=NS)
_PREC = lax.Precision.HIGHEST


def _dot(a, b):
    return jnp.dot(a, b, precision=_PREC, preferred_element_type=f32)


def _sigmoid(x):
    return 1.0 / (1.0 + jnp.exp(-x))


def _seg_sum_body(vals_hbm, idx_hbm, zeros_hbm, out_hbm, idx_v, vals_v, acc, sem):
    c = lax.axis_index("c")
    s = lax.axis_index("s")
    w = c * NS + s
    r0 = s * RPT
    pltpu.sync_copy(zeros_hbm.at[pl.ds(r0, RPT)], acc.at[pl.ds(r0, RPT)])
    plsc.subcore_barrier()

    def body(i, carry):
        off = pl.multiple_of(w * EPW + i * C, 8)
        pltpu.sync_copy(idx_hbm.at[pl.ds(off, C)], idx_v)
        pltpu.async_copy(vals_hbm.at[pl.ds(off, C)], vals_v, sem).wait()
        pltpu.sync_copy(vals_v, acc.at[idx_v], add=True)
        return carry

    lax.fori_loop(0, NCHUNK, body, 0)
    plsc.subcore_barrier()
    pltpu.sync_copy(acc.at[pl.ds(r0, RPT)], out_hbm.at[c, pl.ds(r0, RPT)])


def _seg_sum(vals, idx, zeros):
    return pl.kernel(
        _seg_sum_body,
        out_type=jax.ShapeDtypeStruct((NC, N, HID), f32),
        mesh=_MESH,
        scratch_types=[
            pltpu.VMEM((C,), jnp.int32),
            pltpu.VMEM((C, HID), f32),
            pltpu.VMEM_SHARED((N, HID), f32),
            pltpu.SemaphoreType.DMA,
        ],
    )(vals, idx, zeros)


def _gather_body(table_hbm, idx_hbm, out_hbm, idx_v, rows_v, sem):
    c = lax.axis_index("c")
    s = lax.axis_index("s")
    w = c * NS + s

    def body(i, carry):
        off = pl.multiple_of(w * EPW + i * C, 8)
        pltpu.sync_copy(idx_hbm.at[pl.ds(off, C)], idx_v)
        pltpu.async_copy(table_hbm.at[idx_v], rows_v, sem).wait()
        pltpu.sync_copy(rows_v, out_hbm.at[pl.ds(off, C)])
        return carry

    lax.fori_loop(0, NCHUNK, body, 0)


def _gather(table, idx):
    return pl.kernel(
        _gather_body,
        out_type=jax.ShapeDtypeStruct((E, HID), f32),
        mesh=_MESH,
        scratch_types=[
            pltpu.VMEM((C,), jnp.int32),
            pltpu.VMEM((C, HID), f32),
            pltpu.SemaphoreType.DMA,
        ],
    )(table, idx)


def _gs_body(table_hbm, src_hbm, dst_hbm, zeros_hbm, out_hbm,
             sidx_v, didx_v, rows_v, acc, sem):
    c = lax.axis_index("c")
    s = lax.axis_index("s")
    w = c * NS + s
    r0 = s * RPT
    pltpu.sync_copy(zeros_hbm.at[pl.ds(r0, RPT)], acc.at[pl.ds(r0, RPT)])
    plsc.subcore_barrier()

    def body(i, carry):
        off = pl.multiple_of(w * EPW + i * C, 8)
        pltpu.sync_copy(src_hbm.at[pl.ds(off, C)], sidx_v)
        pltpu.sync_copy(dst_hbm.at[pl.ds(off, C)], didx_v)
        pltpu.async_copy(table_hbm.at[sidx_v], rows_v, sem).wait()
        pltpu.sync_copy(rows_v, acc.at[didx_v], add=True)
        return carry

    lax.fori_loop(0, NCHUNK, body, 0)
    plsc.subcore_barrier()
    pltpu.sync_copy(acc.at[pl.ds(r0, RPT)], out_hbm.at[c, pl.ds(r0, RPT)])


def _gather_seg_sum(table, src, dst, zeros):
    return pl.kernel(
        _gs_body,
        out_type=jax.ShapeDtypeStruct((NC, N, HID), f32),
        mesh=_MESH,
        scratch_types=[
            pltpu.VMEM((C,), jnp.int32),
            pltpu.VMEM((C,), jnp.int32),
            pltpu.VMEM((C, HID), f32),
            pltpu.VMEM_SHARED((N, HID), f32),
            pltpu.SemaphoreType.DMA,
        ],
    )(table, src, dst, zeros)


BN = 1000
GN = N // BN
BE = 2000
GE = E // BE


def _xa_body(wid_ref, emb_ref, wr_ref, urb_ref, x_ref, a_ref):
    wid = wid_ref[0, 0, :]
    cols = lax.broadcasted_iota(jnp.int32, (BN, VOCAB), 1)
    onehot = (cols == wid[:, None]).astype(f32)
    x = _dot(onehot, emb_ref[...])
    x_ref[...] = x
    a_ref[...] = _dot(x, wr_ref[...]) + urb_ref[...]


def _tc_xa(wid3, emb, Wr_w, urb):
    return pl.pallas_call(
        _xa_body,
        grid=(GN,),
        in_specs=[
            pl.BlockSpec((1, 1, BN), lambda i: (i, 0, 0)),
            pl.BlockSpec((VOCAB, HID), lambda i: (0, 0)),
            pl.BlockSpec((HID, HID), lambda i: (0, 0)),
            pl.BlockSpec((1, HID), lambda i: (0, 0)),
        ],
        out_specs=[
            pl.BlockSpec((BN, HID), lambda i: (i, 0)),
            pl.BlockSpec((BN, HID), lambda i: (i, 0)),
        ],
        out_shape=[
            jax.ShapeDtypeStruct((N, HID), f32),
            jax.ShapeDtypeStruct((N, HID), f32),
        ],
    )(wid3, emb, Wr_w, urb)


def _rm_body(m_ref, ad_ref, ur_ref, rm_ref):
    mm = m_ref[...]
    u = _dot(mm, ur_ref[...])
    rm_ref[...] = _sigmoid(ad_ref[...] + u) * mm


def _tc_rm(m, a_dst, Ur_w):
    return pl.pallas_call(
        _rm_body,
        grid=(GE,),
        in_specs=[
            pl.BlockSpec((BE, HID), lambda i: (i, 0)),
            pl.BlockSpec((BE, HID), lambda i: (i, 0)),
            pl.BlockSpec((HID, HID), lambda i: (0, 0)),
        ],
        out_specs=pl.BlockSpec((BE, HID), lambda i: (i, 0)),
        out_shape=jax.ShapeDtypeStruct((E, HID), f32),
    )(m, a_dst, Ur_w)


def _mn_body(x_ref, nmp_ref, nrmp_ref, wz_ref, bz_ref, wh_ref, bh_ref, mn_ref):
    nm = nmp_ref[0] + nmp_ref[1]
    nrm = nrmp_ref[0] + nrmp_ref[1]
    x = x_ref[...]
    wz = wz_ref[...]
    wh = wh_ref[...]
    z = _sigmoid(_dot(x, wz[:HID]) + _dot(nm, wz[HID:]) + bz_ref[...])
    ht = jnp.tanh(_dot(x, wh[:HID]) + _dot(nrm, wh[HID:]) + bh_ref[...])
    mn_ref[...] = (1.0 - z) * nm + z * ht


def _tc_mn(x, nm_p, nrm_p, Wz_w, bz, Wh_w, bh):
    return pl.pallas_call(
        _mn_body,
        grid=(GN,),
        in_specs=[
            pl.BlockSpec((BN, HID), lambda i: (i, 0)),
            pl.BlockSpec((NC, BN, HID), lambda i: (0, i, 0)),
            pl.BlockSpec((NC, BN, HID), lambda i: (0, i, 0)),
            pl.BlockSpec((2 * HID, HID), lambda i: (0, 0)),
            pl.BlockSpec((1, HID), lambda i: (0, 0)),
            pl.BlockSpec((2 * HID, HID), lambda i: (0, 0)),
            pl.BlockSpec((1, HID), lambda i: (0, 0)),
        ],
        out_specs=pl.BlockSpec((BN, HID), lambda i: (i, 0)),
        out_shape=jax.ShapeDtypeStruct((N, HID), f32),
    )(x, nm_p, nrm_p, Wz_w, bz, Wh_w, bh)


def _loss_body(x_ref, hp_ref, tv_ref, wid_ref, pt_ref, ww_ref, wb_ref,
               wo_ref, wob_ref, uw_ref, ub_ref, us_ref, usb_ref, out_ref):
    i = pl.program_id(0)
    h = hp_ref[0] + hp_ref[1]
    x = x_ref[...]
    tv = tv_ref[...]
    ww = ww_ref[...]
    uw = uw_ref[...]
    t1 = jnp.maximum(_dot(h, ww[:HID]) + _dot(tv, ww[HID:]) + wb_ref[...], 0.0)
    q = _dot(t1, wo_ref[...]) + wob_ref[...]
    rmax = jnp.max(q, axis=1, keepdims=True)
    lse = rmax[:, 0] + jnp.log(jnp.sum(jnp.exp(q - rmax), axis=1))
    wid = wid_ref[0, 0, :]
    cols = lax.broadcasted_iota(jnp.int32, (BN, VOCAB), 1)
    picked = jnp.sum(jnp.where(cols == wid[:, None], q, 0.0), axis=1)
    q_loss = jnp.sum(lse - picked)
    first = jnp.min(jnp.where(q == rmax, cols, VOCAB), axis=1)
    q_acc = jnp.sum((first == wid).astype(f32))
    t2 = jnp.maximum(_dot(x, uw[:HID]) + _dot(h, uw[HID:2 * HID])
                     + _dot(tv, uw[2 * HID:]) + ub_ref[...], 0.0)
    p = jnp.sum(t2 * us_ref[...], axis=1) + usb_ref[0, 0]
    pti = pt_ref[0, 0, :]
    pt = pti.astype(f32)
    p_loss = jnp.sum(jnp.maximum(p, 0.0) - p * pt
                     + jnp.log(1.0 + jnp.exp(-jnp.abs(p))))
    p_acc = jnp.sum(((p > 0).astype(jnp.int32) == pti).astype(f32))
    lane = lax.broadcasted_iota(jnp.int32, (1, HID), 1)
    part = (jnp.where(lane == 0, q_loss / 256.0, 0.0)
            + jnp.where(lane == 1, p_loss / 256.0, 0.0)
            + jnp.where(lane == 2, q_acc / N, 0.0)
            + jnp.where(lane == 3, p_acc / N, 0.0))

    @pl.when(i == 0)
    def _init():
        out_ref[...] = part

    @pl.when(i > 0)
    def _accum():
        out_ref[...] += part


def _tc_loss(x, h_p, tree_vec, wid3, pt3, W_w, wb, Wo_w, wob, U_w, ub, us, usb):
    return pl.pallas_call(
        _loss_body,
        grid=(GN,),
        in_specs=[
            pl.BlockSpec((BN, HID), lambda i: (i, 0)),
            pl.BlockSpec((NC, BN, HID), lambda i: (0, i, 0)),
            pl.BlockSpec((BN, LAT), lambda i: (i, 0)),
            pl.BlockSpec((1, 1, BN), lambda i: (i, 0, 0)),
            pl.BlockSpec((1, 1, BN), lambda i: (i, 0, 0)),
            pl.BlockSpec((HID + LAT, HID), lambda i: (0, 0)),
            pl.BlockSpec((1, HID), lambda i: (0, 0)),
            pl.BlockSpec((HID, VOCAB), lambda i: (0, 0)),
            pl.BlockSpec((1, VOCAB), lambda i: (0, 0)),
            pl.BlockSpec((2 * HID + LAT, HID), lambda i: (0, 0)),
            pl.BlockSpec((1, HID), lambda i: (0, 0)),
            pl.BlockSpec((1, HID), lambda i: (0, 0)),
            pl.BlockSpec((1, 1), lambda i: (0, 0)),
        ],
        out_specs=pl.BlockSpec((1, HID), lambda i: (0, 0)),
        out_shape=jax.ShapeDtypeStruct((1, HID), f32),
    )(x, h_p, tree_vec, wid3, pt3, W_w, wb, Wo_w, wob, U_w, ub, us, usb)


def kernel(wid, edge_index, p_targets, tree_vec, m, emb, W_w, W_b, U_w, U_b,
           Wo_w, Wo_b, Us_w, Us_b, Wz_w, Wz_b, Wr_w, Ur_w, Ur_b, Wh_w, Wh_b):
    src = edge_index[0]
    dst = edge_index[1]
    zeros = jnp.zeros((N, HID), f32)
    wid3 = wid.reshape(GN, 1, BN)
    pt3 = p_targets.reshape(GN, 1, BN)

    x, a = _tc_xa(wid3, emb, Wr_w, Ur_b.reshape(1, HID))
    nm_p = _seg_sum(m, dst, zeros)
    a_dst = _gather(a, dst)
    rm = _tc_rm(m, a_dst, Ur_w)
    nrm_p = _seg_sum(rm, dst, zeros)
    mn = _tc_mn(x, nm_p, nrm_p, Wz_w, Wz_b.reshape(1, HID),
                Wh_w, Wh_b.reshape(1, HID))
    h_p = _gather_seg_sum(mn, src, dst, zeros)
    out = _tc_loss(x, h_p, tree_vec, wid3, pt3,
                   W_w, W_b.reshape(1, HID), Wo_w, Wo_b.reshape(1, VOCAB),
                   U_w, U_b.reshape(1, HID), Us_w.reshape(1, HID),
                   Us_b.reshape(1, 1))
    return out[0, :4]


# speedup vs baseline: 3.3877x; 3.3877x over previous
import jax
import jax.numpy as jnp
from jax import lax
from jax.experimental import pallas as pl
from jax.experimental.pallas import tpu as pltpu
from jax.experimental.pallas import tpu_sc as plsc

N = 10000
E = 320000
HID = 128
LAT = 64
VOCAB = 800

NC, NS = 2, 16
NW = NC * NS
EPW = E // NW
C = 80
NCHUNK = EPW // C
RPT = 624
RREM = N - NS * RPT

f32 = jnp.float32
_MESH = plsc.VectorSubcoreMesh(core_axis_name="c", subcore_axis_name="s",
                               num_cores=NC, num_subcores=NS)
_PREC = lax.Precision.HIGHEST


def _dot(a, b):
    return jnp.dot(a, b, precision=_PREC, preferred_element_type=f32)


def _sigmoid(x):
    return 1.0 / (1.0 + jnp.exp(-x))


def _acc_init(zeros_hbm, acc, s):
    r0 = pl.multiple_of(s * RPT, 8)
    pltpu.sync_copy(zeros_hbm.at[pl.ds(r0, RPT)], acc.at[pl.ds(r0, RPT)])

    @pl.when(s == 0)
    def _rem():
        pltpu.sync_copy(zeros_hbm.at[pl.ds(NS * RPT, RREM)],
                        acc.at[pl.ds(NS * RPT, RREM)])


def _acc_write(acc, out_hbm, c, s):
    r0 = pl.multiple_of(s * RPT, 8)
    pltpu.sync_copy(acc.at[pl.ds(r0, RPT)], out_hbm.at[c, pl.ds(r0, RPT)])

    @pl.when(s == 0)
    def _rem():
        pltpu.sync_copy(acc.at[pl.ds(NS * RPT, RREM)],
                        out_hbm.at[c, pl.ds(NS * RPT, RREM)])


def _seg_sum_body(vals_hbm, idx_hbm, zeros_hbm, out_hbm, idx_v, vals_v, acc, sem):
    c = lax.axis_index("c")
    s = lax.axis_index("s")
    w = c * NS + s
    _acc_init(zeros_hbm, acc, s)
    plsc.subcore_barrier()

    def body(i, carry):
        off = pl.multiple_of(w * EPW + i * C, 8)
        pltpu.sync_copy(idx_hbm.at[pl.ds(off, C)], idx_v)
        pltpu.async_copy(vals_hbm.at[pl.ds(off, C)], vals_v, sem).wait()
        pltpu.sync_copy(vals_v, acc.at[idx_v], add=True)
        return carry

    lax.fori_loop(0, NCHUNK, body, 0)
    plsc.subcore_barrier()
    _acc_write(acc, out_hbm, c, s)


def _seg_sum(vals, idx, zeros):
    return pl.kernel(
        _seg_sum_body,
        out_type=jax.ShapeDtypeStruct((NC, N, HID), f32),
        mesh=_MESH,
        scratch_types=[
            pltpu.VMEM((C,), jnp.int32),
            pltpu.VMEM((C, HID), f32),
            pltpu.VMEM_SHARED((N, HID), f32),
            pltpu.SemaphoreType.DMA,
        ],
    )(vals, idx, zeros)


def _gather_body(table_hbm, idx_hbm, out_hbm, idx_v, rows_v, sem):
    c = lax.axis_index("c")
    s = lax.axis_index("s")
    w = c * NS + s

    def body(i, carry):
        off = pl.multiple_of(w * EPW + i * C, 8)
        pltpu.sync_copy(idx_hbm.at[pl.ds(off, C)], idx_v)
        pltpu.async_copy(table_hbm.at[idx_v], rows_v, sem).wait()
        pltpu.sync_copy(rows_v, out_hbm.at[pl.ds(off, C)])
        return carry

    lax.fori_loop(0, NCHUNK, body, 0)


def _gather(table, idx):
    return pl.kernel(
        _gather_body,
        out_type=jax.ShapeDtypeStruct((E, HID), f32),
        mesh=_MESH,
        scratch_types=[
            pltpu.VMEM((C,), jnp.int32),
            pltpu.VMEM((C, HID), f32),
            pltpu.SemaphoreType.DMA,
        ],
    )(table, idx)


def _gs_body(table_hbm, src_hbm, dst_hbm, zeros_hbm, out_hbm,
             sidx_v, didx_v, rows_v, acc, sem):
    c = lax.axis_index("c")
    s = lax.axis_index("s")
    w = c * NS + s
    _acc_init(zeros_hbm, acc, s)
    plsc.subcore_barrier()

    def body(i, carry):
        off = pl.multiple_of(w * EPW + i * C, 8)
        pltpu.sync_copy(src_hbm.at[pl.ds(off, C)], sidx_v)
        pltpu.sync_copy(dst_hbm.at[pl.ds(off, C)], didx_v)
        pltpu.async_copy(table_hbm.at[sidx_v], rows_v, sem).wait()
        pltpu.sync_copy(rows_v, acc.at[didx_v], add=True)
        return carry

    lax.fori_loop(0, NCHUNK, body, 0)
    plsc.subcore_barrier()
    _acc_write(acc, out_hbm, c, s)


def _gather_seg_sum(table, src, dst, zeros):
    return pl.kernel(
        _gs_body,
        out_type=jax.ShapeDtypeStruct((NC, N, HID), f32),
        mesh=_MESH,
        scratch_types=[
            pltpu.VMEM((C,), jnp.int32),
            pltpu.VMEM((C,), jnp.int32),
            pltpu.VMEM((C, HID), f32),
            pltpu.VMEM_SHARED((N, HID), f32),
            pltpu.SemaphoreType.DMA,
        ],
    )(table, src, dst, zeros)


BN = 1000
GN = N // BN
BE = 2000
GE = E // BE


def _xa_body(wid_ref, emb_ref, wr_ref, urb_ref, x_ref, a_ref):
    wid = wid_ref[0, 0, :]
    cols = lax.broadcasted_iota(jnp.int32, (BN, VOCAB), 1)
    onehot = (cols == wid[:, None]).astype(f32)
    x = _dot(onehot, emb_ref[...])
    x_ref[...] = x
    a_ref[...] = _dot(x, wr_ref[...]) + urb_ref[...]


def _tc_xa(wid3, emb, Wr_w, urb):
    return pl.pallas_call(
        _xa_body,
        grid=(GN,),
        in_specs=[
            pl.BlockSpec((1, 1, BN), lambda i: (i, 0, 0)),
            pl.BlockSpec((VOCAB, HID), lambda i: (0, 0)),
            pl.BlockSpec((HID, HID), lambda i: (0, 0)),
            pl.BlockSpec((1, HID), lambda i: (0, 0)),
        ],
        out_specs=[
            pl.BlockSpec((BN, HID), lambda i: (i, 0)),
            pl.BlockSpec((BN, HID), lambda i: (i, 0)),
        ],
        out_shape=[
            jax.ShapeDtypeStruct((N, HID), f32),
            jax.ShapeDtypeStruct((N, HID), f32),
        ],
    )(wid3, emb, Wr_w, urb)


def _rm_body(m_ref, ad_ref, ur_ref, rm_ref):
    mm = m_ref[...]
    u = _dot(mm, ur_ref[...])
    rm_ref[...] = _sigmoid(ad_ref[...] + u) * mm


def _tc_rm(m, a_dst, Ur_w):
    return pl.pallas_call(
        _rm_body,
        grid=(GE,),
        in_specs=[
            pl.BlockSpec((BE, HID), lambda i: (i, 0)),
            pl.BlockSpec((BE, HID), lambda i: (i, 0)),
            pl.BlockSpec((HID, HID), lambda i: (0, 0)),
        ],
        out_specs=pl.BlockSpec((BE, HID), lambda i: (i, 0)),
        out_shape=jax.ShapeDtypeStruct((E, HID), f32),
    )(m, a_dst, Ur_w)


def _mn_body(x_ref, nmp_ref, nrmp_ref, wz_ref, bz_ref, wh_ref, bh_ref, mn_ref):
    nm = nmp_ref[0] + nmp_ref[1]
    nrm = nrmp_ref[0] + nrmp_ref[1]
    x = x_ref[...]
    wz = wz_ref[...]
    wh = wh_ref[...]
    z = _sigmoid(_dot(x, wz[:HID]) + _dot(nm, wz[HID:]) + bz_ref[...])
    ht = jnp.tanh(_dot(x, wh[:HID]) + _dot(nrm, wh[HID:]) + bh_ref[...])
    mn_ref[...] = (1.0 - z) * nm + z * ht


def _tc_mn(x, nm_p, nrm_p, Wz_w, bz, Wh_w, bh):
    return pl.pallas_call(
        _mn_body,
        grid=(GN,),
        in_specs=[
            pl.BlockSpec((BN, HID), lambda i: (i, 0)),
            pl.BlockSpec((NC, BN, HID), lambda i: (0, i, 0)),
            pl.BlockSpec((NC, BN, HID), lambda i: (0, i, 0)),
            pl.BlockSpec((2 * HID, HID), lambda i: (0, 0)),
            pl.BlockSpec((1, HID), lambda i: (0, 0)),
            pl.BlockSpec((2 * HID, HID), lambda i: (0, 0)),
            pl.BlockSpec((1, HID), lambda i: (0, 0)),
        ],
        out_specs=pl.BlockSpec((BN, HID), lambda i: (i, 0)),
        out_shape=jax.ShapeDtypeStruct((N, HID), f32),
    )(x, nm_p, nrm_p, Wz_w, bz, Wh_w, bh)


def _loss_body(x_ref, hp_ref, tv_ref, wid_ref, pt_ref, ww_ref, wb_ref,
               wo_ref, wob_ref, uw_ref, ub_ref, us_ref, usb_ref, out_ref):
    i = pl.program_id(0)
    h = hp_ref[0] + hp_ref[1]
    x = x_ref[...]
    tv = tv_ref[...]
    ww = ww_ref[...]
    uw = uw_ref[...]
    t1 = jnp.maximum(_dot(h, ww[:HID]) + _dot(tv, ww[HID:]) + wb_ref[...], 0.0)
    q = _dot(t1, wo_ref[...]) + wob_ref[...]
    rmax = jnp.max(q, axis=1, keepdims=True)
    lse = rmax[:, 0] + jnp.log(jnp.sum(jnp.exp(q - rmax), axis=1))
    wid = wid_ref[0, 0, :]
    cols = lax.broadcasted_iota(jnp.int32, (BN, VOCAB), 1)
    picked = jnp.sum(jnp.where(cols == wid[:, None], q, 0.0), axis=1)
    q_loss = jnp.sum(lse - picked)
    first = jnp.min(jnp.where(q == rmax, cols, VOCAB), axis=1)
    q_acc = jnp.sum((first == wid).astype(f32))
    t2 = jnp.maximum(_dot(x, uw[:HID]) + _dot(h, uw[HID:2 * HID])
                     + _dot(tv, uw[2 * HID:]) + ub_ref[...], 0.0)
    p = jnp.sum(t2 * us_ref[...], axis=1) + usb_ref[0, 0]
    pti = pt_ref[0, 0, :]
    pt = pti.astype(f32)
    p_loss = jnp.sum(jnp.maximum(p, 0.0) - p * pt
                     + jnp.log(1.0 + jnp.exp(-jnp.abs(p))))
    p_acc = jnp.sum(((p > 0).astype(jnp.int32) == pti).astype(f32))
    lane = lax.broadcasted_iota(jnp.int32, (1, HID), 1)
    part = (jnp.where(lane == 0, q_loss / 256.0, 0.0)
            + jnp.where(lane == 1, p_loss / 256.0, 0.0)
            + jnp.where(lane == 2, q_acc / N, 0.0)
            + jnp.where(lane == 3, p_acc / N, 0.0))

    @pl.when(i == 0)
    def _init():
        out_ref[...] = part

    @pl.when(i > 0)
    def _accum():
        out_ref[...] += part


def _tc_loss(x, h_p, tree_vec, wid3, pt3, W_w, wb, Wo_w, wob, U_w, ub, us, usb):
    return pl.pallas_call(
        _loss_body,
        grid=(GN,),
        in_specs=[
            pl.BlockSpec((BN, HID), lambda i: (i, 0)),
            pl.BlockSpec((NC, BN, HID), lambda i: (0, i, 0)),
            pl.BlockSpec((BN, LAT), lambda i: (i, 0)),
            pl.BlockSpec((1, 1, BN), lambda i: (i, 0, 0)),
            pl.BlockSpec((1, 1, BN), lambda i: (i, 0, 0)),
            pl.BlockSpec((HID + LAT, HID), lambda i: (0, 0)),
            pl.BlockSpec((1, HID), lambda i: (0, 0)),
            pl.BlockSpec((HID, VOCAB), lambda i: (0, 0)),
            pl.BlockSpec((1, VOCAB), lambda i: (0, 0)),
            pl.BlockSpec((2 * HID + LAT, HID), lambda i: (0, 0)),
            pl.BlockSpec((1, HID), lambda i: (0, 0)),
            pl.BlockSpec((1, HID), lambda i: (0, 0)),
            pl.BlockSpec((1, 1), lambda i: (0, 0)),
        ],
        out_specs=pl.BlockSpec((1, HID), lambda i: (0, 0)),
        out_shape=jax.ShapeDtypeStruct((1, HID), f32),
    )(x, h_p, tree_vec, wid3, pt3, W_w, wb, Wo_w, wob, U_w, ub, us, usb)


def kernel(wid, edge_index, p_targets, tree_vec, m, emb, W_w, W_b, U_w, U_b,
           Wo_w, Wo_b, Us_w, Us_b, Wz_w, Wz_b, Wr_w, Ur_w, Ur_b, Wh_w, Wh_b):
    src = edge_index[0]
    dst = edge_index[1]
    zeros = jnp.zeros((N, HID), f32)
    wid3 = wid.reshape(GN, 1, BN)
    pt3 = p_targets.reshape(GN, 1, BN)

    x, a = _tc_xa(wid3, emb, Wr_w, Ur_b.reshape(1, HID))
    nm_p = _seg_sum(m, dst, zeros)
    a_dst = _gather(a, dst)
    rm = _tc_rm(m, a_dst, Ur_w)
    nrm_p = _seg_sum(rm, dst, zeros)
    mn = _tc_mn(x, nm_p, nrm_p, Wz_w, Wz_b.reshape(1, HID),
                Wh_w, Wh_b.reshape(1, HID))
    h_p = _gather_seg_sum(mn, src, dst, zeros)
    out = _tc_loss(x, h_p, tree_vec, wid3, pt3,
                   W_w, W_b.reshape(1, HID), Wo_w, Wo_b.reshape(1, VOCAB),
                   U_w, U_b.reshape(1, HID), Us_w.reshape(1, HID),
                   Us_b.reshape(1, 1))
    return out[0, :4]
